# baseline (speedup 1.0000x reference)
import math

import jax
import jax.numpy as jnp
from jax import lax
from jax.experimental import pallas as pl

_MAX_RATIO = abs(math.log(16.0 / 1000.0))


def _roll_lanes(x, shift):
    return jnp.concatenate([x[:, -shift:], x[:, :-shift]], axis=1)


def _decode_body(d_ref, a_ref, out_ref):
    a = a_ref[...]
    d = d_ref[...]
    a2 = _roll_lanes(a, -2)
    d2 = _roll_lanes(d, -2)
    wh = a2 - a
    cxy = a + 0.5 * wh
    ncxy = cxy + d * wh
    hnwh = 0.5 * wh * jnp.exp(jnp.clip(d2, -_MAX_RATIO, _MAX_RATIO))
    lane = lax.broadcasted_iota(jnp.int32, a.shape, 1)
    is_lo = (lane & 2) == 0
    out = jnp.where(is_lo, ncxy - hnwh, _roll_lanes(ncxy + hnwh, 2))
    out_ref[...] = jnp.clip(out, 0.0, 1.0)


def kernel(logits, regress, anchors):
    n = regress.shape[1]
    rows = n * 4 // 128
    df = jnp.reshape(regress, (rows, 128))
    af = jnp.reshape(anchors, (rows, 128))
    boxes = pl.pallas_call(
        _decode_body,
        out_shape=jax.ShapeDtypeStruct((rows, 128), regress.dtype),
    )(df, af)
    return (logits, jnp.reshape(boxes, regress.shape))
